# baseline (speedup 1.0000x reference)
import functools

import jax
import jax.numpy as jnp
from jax import lax
from jax.experimental import pallas as pl
from jax.experimental.pallas import tpu as pltpu
from jax.experimental.pallas import tpu_sc as plsc

P = 102928
N = 256
TABLE = 25000000
CLIP = 40.0

NC, NS = 2, 16
NW = NC * NS
C = 3232
CH = C // 16
BUF = 3248

PD = 808 * 128
NPAD = N + 16
G = 16
NSC = 160
WROWS = 806
WTC = WROWS * 128
OROWS = 805
ACAP = (TABLE - WTC) // 128
THRESH = ACAP * 128 + 128
NT = N - NSC
NPAIR = NSC // (2 * G)


def _coeff_body(ret_ref, retT_ref, sc_ref, pos_ref, c_ref, s0_ref):
    ii = lax.broadcasted_iota(jnp.int32, (N, N), 0)
    jj = lax.broadcasted_iota(jnp.int32, (N, N), 1)
    ranks = []
    for k in (0, 1):
        a = -ret_ref[:, k:k + 1]
        r = jnp.zeros((N, 1), jnp.float32)
        for l in (0, 1):
            b = -retT_ref[l:l + 1, :]
            lt = b < a
            eq = b == a
            flat_lt = (2 * jj + l) < (2 * ii + k)
            cnt = jnp.where(lt | (eq & flat_lt), 1.0, 0.0)
            r = r + jnp.sum(cnt, axis=1, keepdims=True)
        ranks.append(r)
    centered0 = ranks[0] / (2.0 * N - 1.0) - 0.5
    centered1 = ranks[1] / (2.0 * N - 1.0) - 0.5
    w = centered0 - centered1
    scales = sc_ref[:]
    c_ref[:] = w * scales * (1.0 / (2.0 * N))
    sign = 2.0 * pos_ref[0, 0] - 1.0
    s0_ref[:] = (sign * scales)[0:16, 0:1]


def _clip_body(x_ref, y_ref, o_ref):
    x = x_ref[:] + y_ref[:]
    gnorm = jnp.sqrt(jnp.sum(x * x))
    o_ref[:] = x * (CLIP / jnp.maximum(gnorm, CLIP))


def _tc_gather_body(sd_ref, c_ref, tbl_ref, out_ref, bufa, bufb, sems):
    step = pl.program_id(0)
    nstep = pl.num_programs(0)
    i0 = 2 * step

    def fire(i, buf, slot):
        a = jnp.minimum(sd_ref[i] // 128, ACAP)
        pltpu.make_async_copy(
            tbl_ref.at[pl.ds(a * 128, WTC)], buf, sems.at[slot]).start()

    def accum(i, buf, slot):
        pltpu.make_async_copy(
            tbl_ref.at[pl.ds(0, WTC)], buf, sems.at[slot]).wait()
        s = sd_ref[i]
        r = s - jnp.minimum(s // 128, ACAP) * 128
        wt = jnp.where(s < THRESH, c_ref[i], 0.0)
        v = buf[:].reshape(WROWS, 128)
        rolled = pltpu.roll(v, (128 - r) % 128, axis=1)
        lane = lax.broadcasted_iota(jnp.int32, (OROWS, 128), 1)
        contrib = jnp.where(lane < 128 - r, rolled[0:OROWS, :],
                            rolled[1:WROWS, :])
        out_ref[pl.ds(0, OROWS), :] = (
            out_ref[pl.ds(0, OROWS), :] + wt * contrib)

    @pl.when(step == 0)
    def _():
        out_ref[:] = jnp.zeros_like(out_ref)
        fire(0, bufa, 0)

    fire(i0 + 1, bufb, 1)
    accum(i0, bufa, 0)

    @pl.when(step < nstep - 1)
    def _():
        fire(i0 + 2, bufa, 0)

    accum(i0 + 1, bufb, 1)

    @pl.when(step == nstep - 1)
    def _():
        lanes = lax.broadcasted_iota(jnp.int32, (1, 128), 1)
        row = P // 128
        out_ref[pl.ds(row, 1), :] = jnp.where(
            lanes < P - row * 128, out_ref[pl.ds(row, 1), :], 0.0)


def _sc_body(tbl, prm, sd, cvec, s0, dlt, prt, *scr):
    sv, cv, s0v = scr[0], scr[1], scr[2]
    bufA = list(scr[3:3 + G])
    bufB = list(scr[3 + G:3 + 2 * G])
    pbuf0, acc, pb, semA, semB, semP = scr[3 + 2 * G:]
    cid = lax.axis_index("c")
    sid = lax.axis_index("s")
    wid = cid * NS + sid
    lo = jnp.minimum(wid * C, P - C)

    pltpu.sync_copy(sd, sv.at[pl.ds(0, N)])
    pltpu.sync_copy(cvec, cv.at[pl.ds(0, N)])
    pltpu.sync_copy(s0, s0v)
    pltpu.sync_copy(prm.at[pl.ds(lo, C)], pb)

    def window(i):
        off = sv[pl.ds(i, 16)][0] + lo
        start = jnp.minimum((off // 16) * 16, TABLE - BUF)
        return start, off - start

    def coeff(i):
        return cv[pl.ds(i, 16)][0]

    def fire(base_i, bufref, sem):
        for u in range(G):
            start, _ = window(base_i + u)
            pltpu.make_async_copy(
                tbl.at[pl.ds(start, BUF)], bufref[u], sem).start()

    def drain(bufref, sem):
        for u in range(G):
            pltpu.make_async_copy(
                tbl.at[pl.ds(0, BUF)], bufref[u], sem).wait()

    def compute(base_i, bufref):
        dd = []
        cc = []
        for u in range(G):
            _, d = window(base_i + u)
            dd.append(d)
            cc.append(coeff(base_i + u))

        @plsc.parallel_loop(0, CH, unroll=4)
        def body(k):
            b = k * 16
            m = [cc[u] * bufref[u][pl.ds(dd[u] + b, 16)] for u in range(G)]
            while len(m) > 1:
                m = [m[2 * v] + m[2 * v + 1] for v in range(len(m) // 2)]
            acc[pl.ds(b, 16)] = acc[pl.ds(b, 16)] + m[0]

    fire(0, bufA, semA)
    start0, d0 = window(0)
    pltpu.make_async_copy(tbl.at[pl.ds(start0, BUF)], pbuf0, semP).start()
    sg = s0v[:][0]
    pltpu.make_async_copy(tbl.at[pl.ds(0, BUF)], pbuf0, semP).wait()

    @plsc.parallel_loop(0, CH, unroll=4)
    def init_chunk(k):
        b = k * 16
        acc[pl.ds(b, 16)] = jnp.zeros((16,), jnp.float32)
        pb[pl.ds(b, 16)] = pb[pl.ds(b, 16)] + sg * pbuf0[pl.ds(d0 + b, 16)]

    def pair(g, _):
        base = 2 * G * g
        fire(base + G, bufB, semB)
        drain(bufA, semA)
        compute(base, bufA)

        @pl.when(g < NPAIR - 1)
        def _():
            fire(base + 2 * G, bufA, semA)

        drain(bufB, semB)
        compute(base + G, bufB)
        return 0

    lax.fori_loop(0, NPAIR, pair, 0)

    def rare(i, _):
        s = sv[pl.ds(i, 16)][0]

        @pl.when(s >= THRESH)
        def _():
            off = s + lo
            start = jnp.minimum((off // 16) * 16, TABLE - BUF)
            pltpu.sync_copy(tbl.at[pl.ds(start, BUF)], pbuf0)
            d = off - start
            ci = cv[pl.ds(i, 16)][0]

            @plsc.parallel_loop(0, CH, unroll=4)
            def chunk(k):
                b = k * 16
                acc[pl.ds(b, 16)] = (
                    acc[pl.ds(b, 16)] + ci * pbuf0[pl.ds(d + b, 16)])

        return 0

    lax.fori_loop(NSC, N, rare, 0)

    pltpu.sync_copy(acc, dlt.at[pl.ds(lo, C)])
    pltpu.sync_copy(pb, prt.at[pl.ds(lo, C)])

    @pl.when(wid == NW - 1)
    def _():
        @plsc.parallel_loop(0, (PD - P) // 16, unroll=2)
        def zchunk(k):
            pbuf0[pl.ds(k * 16, 16)] = jnp.zeros((16,), jnp.float32)

        pltpu.sync_copy(pbuf0.at[pl.ds(0, PD - P)], dlt.at[pl.ds(P, PD - P)])


def kernel(noise_table, params, perturbation_seeds, returns,
           perturbation_scales, positive_perturbation):
    returns = returns.astype(jnp.float32)
    scales_col = perturbation_scales.astype(jnp.float32).reshape(N, 1)
    pos = jnp.asarray(positive_perturbation, jnp.float32).reshape(1, 1)

    cvec_col, s0_col = pl.pallas_call(
        _coeff_body,
        out_shape=(
            jax.ShapeDtypeStruct((N, 1), jnp.float32),
            jax.ShapeDtypeStruct((16, 1), jnp.float32),
        ),
        in_specs=[
            pl.BlockSpec(memory_space=pltpu.VMEM),
            pl.BlockSpec(memory_space=pltpu.VMEM),
            pl.BlockSpec(memory_space=pltpu.VMEM),
            pl.BlockSpec(memory_space=pltpu.SMEM),
        ],
        out_specs=(
            pl.BlockSpec(memory_space=pltpu.VMEM),
            pl.BlockSpec(memory_space=pltpu.VMEM),
        ),
    )(returns, returns.T, scales_col, pos)

    cvec = cvec_col.reshape(N)
    s0 = s0_col.reshape(16)

    mesh = plsc.VectorSubcoreMesh(
        core_axis_name="c", subcore_axis_name="s",
        num_cores=NC, num_subcores=NS)

    sc_main = pl.kernel(
        _sc_body,
        out_type=(
            jax.ShapeDtypeStruct((PD,), jnp.float32),
            jax.ShapeDtypeStruct((P,), jnp.float32),
        ),
        mesh=mesh,
        scratch_types=[
            pltpu.VMEM((NPAD,), jnp.int32),
            pltpu.VMEM((NPAD,), jnp.float32),
            pltpu.VMEM((16,), jnp.float32),
            *[pltpu.VMEM((BUF,), jnp.float32) for _ in range(2 * G)],
            pltpu.VMEM((BUF,), jnp.float32),
            pltpu.VMEM((C,), jnp.float32),
            pltpu.VMEM((C,), jnp.float32),
            pltpu.SemaphoreType.DMA,
            pltpu.SemaphoreType.DMA,
            pltpu.SemaphoreType.DMA,
        ],
    )
    delta_tc = pl.pallas_call(
        _tc_gather_body,
        grid=(NT // 2,),
        out_shape=jax.ShapeDtypeStruct((PD // 128, 128), jnp.float32),
        in_specs=[
            pl.BlockSpec(memory_space=pltpu.SMEM),
            pl.BlockSpec(memory_space=pltpu.SMEM),
            pl.BlockSpec(memory_space=pl.ANY),
        ],
        out_specs=pl.BlockSpec((PD // 128, 128), lambda i: (0, 0)),
        scratch_shapes=[
            pltpu.VMEM((WTC,), jnp.float32),
            pltpu.VMEM((WTC,), jnp.float32),
            pltpu.SemaphoreType.DMA((2,)),
        ],
    )(perturbation_seeds[NSC:].astype(jnp.int32), cvec[NSC:], noise_table)

    delta_raw, perturbed = sc_main(
        noise_table, params, perturbation_seeds.astype(jnp.int32), cvec, s0)

    dclip = pl.pallas_call(
        _clip_body,
        out_shape=jax.ShapeDtypeStruct((PD // 128, 128), jnp.float32),
    )(delta_raw.reshape(PD // 128, 128), delta_tc)
    delta = dclip.reshape(PD)[:P]
    return delta, perturbed


# speedup vs baseline: 1.2868x; 1.2868x over previous
import functools

import jax
import jax.numpy as jnp
from jax import lax
from jax.experimental import pallas as pl
from jax.experimental.pallas import tpu as pltpu
from jax.experimental.pallas import tpu_sc as plsc

P = 102928
N = 256
TABLE = 25000000
CLIP = 40.0

NC, NS = 2, 16
NW = NC * NS
C = 3232
CH = C // 16
BUF = 3248

PD = 808 * 128
NPAD = N + 16
G = 16
NPAIR = N // (2 * G)


def _coeff_body(ret_ref, retT_ref, sc_ref, pos_ref, c_ref, s0_ref):
    ii = lax.broadcasted_iota(jnp.int32, (N, N), 0)
    jj = lax.broadcasted_iota(jnp.int32, (N, N), 1)
    ranks = []
    for k in (0, 1):
        a = -ret_ref[:, k:k + 1]
        r = jnp.zeros((N, 1), jnp.float32)
        for l in (0, 1):
            b = -retT_ref[l:l + 1, :]
            lt = b < a
            eq = b == a
            flat_lt = (2 * jj + l) < (2 * ii + k)
            cnt = jnp.where(lt | (eq & flat_lt), 1.0, 0.0)
            r = r + jnp.sum(cnt, axis=1, keepdims=True)
        ranks.append(r)
    centered0 = ranks[0] / (2.0 * N - 1.0) - 0.5
    centered1 = ranks[1] / (2.0 * N - 1.0) - 0.5
    w = centered0 - centered1
    scales = sc_ref[:]
    c_ref[:] = w * scales * (1.0 / (2.0 * N))
    sign = 2.0 * pos_ref[0, 0] - 1.0
    s0_ref[:] = (sign * scales)[0:16, 0:1]


def _clip_body(x_ref, o_ref):
    x = x_ref[:]
    gnorm = jnp.sqrt(jnp.sum(x * x))
    o_ref[:] = x * (CLIP / jnp.maximum(gnorm, CLIP))


def _sc_body(tbl, prm, sd, cvec, s0, dlt, prt, *scr):
    sv, cv, s0v = scr[0], scr[1], scr[2]
    bufA = list(scr[3:3 + G])
    bufB = list(scr[3 + G:3 + 2 * G])
    pbuf0, acc, pb, semA, semB, semP = scr[3 + 2 * G:]
    cid = lax.axis_index("c")
    sid = lax.axis_index("s")
    wid = cid * NS + sid
    lo = jnp.minimum(wid * C, P - C)

    pltpu.sync_copy(sd, sv.at[pl.ds(0, N)])
    pltpu.sync_copy(cvec, cv.at[pl.ds(0, N)])
    pltpu.sync_copy(s0, s0v)
    pltpu.sync_copy(prm.at[pl.ds(lo, C)], pb)

    def window(i):
        off = sv[pl.ds(i, 16)][0] + lo
        start = jnp.minimum((off // 16) * 16, TABLE - BUF)
        return start, off - start

    def coeff(i):
        return cv[pl.ds(i, 16)][0]

    def fire(base_i, bufref, sem):
        for u in range(G):
            start, _ = window(base_i + u)
            pltpu.make_async_copy(
                tbl.at[pl.ds(start, BUF)], bufref[u], sem).start()

    def drain(bufref, sem):
        for u in range(G):
            pltpu.make_async_copy(
                tbl.at[pl.ds(0, BUF)], bufref[u], sem).wait()

    def compute(base_i, bufref):
        dd = []
        cc = []
        for u in range(G):
            _, d = window(base_i + u)
            dd.append(d)
            cc.append(coeff(base_i + u))

        @plsc.parallel_loop(0, CH, unroll=4)
        def body(k):
            b = k * 16
            m = [cc[u] * bufref[u][pl.ds(dd[u] + b, 16)] for u in range(G)]
            while len(m) > 1:
                m = [m[2 * v] + m[2 * v + 1] for v in range(len(m) // 2)]
            acc[pl.ds(b, 16)] = acc[pl.ds(b, 16)] + m[0]

    fire(0, bufA, semA)
    start0, d0 = window(0)
    pltpu.make_async_copy(tbl.at[pl.ds(start0, BUF)], pbuf0, semP).start()
    sg = s0v[:][0]
    pltpu.make_async_copy(tbl.at[pl.ds(0, BUF)], pbuf0, semP).wait()

    @plsc.parallel_loop(0, CH, unroll=4)
    def init_chunk(k):
        b = k * 16
        acc[pl.ds(b, 16)] = jnp.zeros((16,), jnp.float32)
        pb[pl.ds(b, 16)] = pb[pl.ds(b, 16)] + sg * pbuf0[pl.ds(d0 + b, 16)]

    def pair(g, _):
        base = 2 * G * g
        fire(base + G, bufB, semB)
        drain(bufA, semA)
        compute(base, bufA)

        @pl.when(g < NPAIR - 1)
        def _():
            fire(base + 2 * G, bufA, semA)

        drain(bufB, semB)
        compute(base + G, bufB)
        return 0

    lax.fori_loop(0, NPAIR, pair, 0)

    pltpu.sync_copy(acc, dlt.at[pl.ds(lo, C)])
    pltpu.sync_copy(pb, prt.at[pl.ds(lo, C)])

    @pl.when(wid == NW - 1)
    def _():
        @plsc.parallel_loop(0, (PD - P) // 16, unroll=2)
        def zchunk(k):
            pbuf0[pl.ds(k * 16, 16)] = jnp.zeros((16,), jnp.float32)

        pltpu.sync_copy(pbuf0.at[pl.ds(0, PD - P)], dlt.at[pl.ds(P, PD - P)])


def kernel(noise_table, params, perturbation_seeds, returns,
           perturbation_scales, positive_perturbation):
    returns = returns.astype(jnp.float32)
    scales_col = perturbation_scales.astype(jnp.float32).reshape(N, 1)
    pos = jnp.asarray(positive_perturbation, jnp.float32).reshape(1, 1)

    cvec_col, s0_col = pl.pallas_call(
        _coeff_body,
        out_shape=(
            jax.ShapeDtypeStruct((N, 1), jnp.float32),
            jax.ShapeDtypeStruct((16, 1), jnp.float32),
        ),
        in_specs=[
            pl.BlockSpec(memory_space=pltpu.VMEM),
            pl.BlockSpec(memory_space=pltpu.VMEM),
            pl.BlockSpec(memory_space=pltpu.VMEM),
            pl.BlockSpec(memory_space=pltpu.SMEM),
        ],
        out_specs=(
            pl.BlockSpec(memory_space=pltpu.VMEM),
            pl.BlockSpec(memory_space=pltpu.VMEM),
        ),
    )(returns, returns.T, scales_col, pos)

    cvec = cvec_col.reshape(N)
    s0 = s0_col.reshape(16)

    mesh = plsc.VectorSubcoreMesh(
        core_axis_name="c", subcore_axis_name="s",
        num_cores=NC, num_subcores=NS)

    sc_main = pl.kernel(
        _sc_body,
        out_type=(
            jax.ShapeDtypeStruct((PD,), jnp.float32),
            jax.ShapeDtypeStruct((P,), jnp.float32),
        ),
        mesh=mesh,
        scratch_types=[
            pltpu.VMEM((NPAD,), jnp.int32),
            pltpu.VMEM((NPAD,), jnp.float32),
            pltpu.VMEM((16,), jnp.float32),
            *[pltpu.VMEM((BUF,), jnp.float32) for _ in range(2 * G)],
            pltpu.VMEM((BUF,), jnp.float32),
            pltpu.VMEM((C,), jnp.float32),
            pltpu.VMEM((C,), jnp.float32),
            pltpu.SemaphoreType.DMA,
            pltpu.SemaphoreType.DMA,
            pltpu.SemaphoreType.DMA,
        ],
    )
    delta_raw, perturbed = sc_main(
        noise_table, params, perturbation_seeds.astype(jnp.int32), cvec, s0)

    dpad = delta_raw.reshape(PD // 128, 128)
    dclip = pl.pallas_call(
        _clip_body,
        out_shape=jax.ShapeDtypeStruct((PD // 128, 128), jnp.float32),
    )(dpad)
    delta = dclip.reshape(PD)[:P]
    return delta, perturbed
